# initial kernel scaffold (unmeasured)
import jax
import jax.numpy as jnp
from jax import lax
from jax.experimental import pallas as pl
from jax.experimental.pallas import tpu as pltpu


def kernel(
    x,
):
    def body(*refs):
        pass

    out_shape = jax.ShapeDtypeStruct(..., jnp.float32)
    return pl.pallas_call(body, out_shape=out_shape)(...)


# baseline (device time: 2340595 ns/iter reference)
import jax
import jax.numpy as jnp
from jax import lax
from jax.experimental import pallas as pl
from jax.experimental.pallas import tpu as pltpu

N_Z = 4


def kernel(x):
    m_per, n = x.shape
    half = m_per // 2

    def body(x_ref, out_ref, copy_sem, send_a, recv_a, send_b, recv_b):
        my_x = lax.axis_index("x")
        my_y = lax.axis_index("y")
        my_z = lax.axis_index("z")
        up = (my_z + 1) % N_Z
        down = (my_z - 1) % N_Z

        barrier = pltpu.get_barrier_semaphore()
        for nz in (up, down):
            pl.semaphore_signal(
                barrier, inc=1,
                device_id=(my_x, my_y, nz),
                device_id_type=pl.DeviceIdType.MESH,
            )
        pl.semaphore_wait(barrier, 2)

        local = pltpu.make_async_copy(
            x_ref, out_ref.at[pl.ds(my_z * m_per, m_per), :], copy_sem
        )
        local.start()

        for h in range(N_Z - 1):
            a_org = (my_z - h) % N_Z
            b_org = (my_z + h) % N_Z
            if h == 0:
                src_a = x_ref.at[pl.ds(0, half), :]
                src_b = x_ref.at[pl.ds(half, half), :]
            else:
                src_a = out_ref.at[pl.ds(a_org * m_per, half), :]
                src_b = out_ref.at[pl.ds(b_org * m_per + half, half), :]

            rdma_a = pltpu.make_async_remote_copy(
                src_ref=src_a,
                dst_ref=out_ref.at[pl.ds(a_org * m_per, half), :],
                send_sem=send_a.at[h],
                recv_sem=recv_a.at[h],
                device_id=(my_x, my_y, up),
                device_id_type=pl.DeviceIdType.MESH,
            )
            rdma_b = pltpu.make_async_remote_copy(
                src_ref=src_b,
                dst_ref=out_ref.at[pl.ds(b_org * m_per + half, half), :],
                send_sem=send_b.at[h],
                recv_sem=recv_b.at[h],
                device_id=(my_x, my_y, down),
                device_id_type=pl.DeviceIdType.MESH,
            )
            rdma_a.start()
            rdma_b.start()
            rdma_a.wait()
            rdma_b.wait()

        local.wait()

    out_shape = jax.ShapeDtypeStruct((N_Z * m_per, n), x.dtype)
    return pl.pallas_call(
        body,
        out_shape=out_shape,
        in_specs=[pl.BlockSpec(memory_space=pl.ANY)],
        out_specs=pl.BlockSpec(memory_space=pl.ANY),
        scratch_shapes=[
            pltpu.SemaphoreType.DMA,
            pltpu.SemaphoreType.DMA((N_Z - 1,)),
            pltpu.SemaphoreType.DMA((N_Z - 1,)),
            pltpu.SemaphoreType.DMA((N_Z - 1,)),
            pltpu.SemaphoreType.DMA((N_Z - 1,)),
        ],
        compiler_params=pltpu.CompilerParams(collective_id=0),
    )(x)


# device time: 2216428 ns/iter; 1.0560x vs baseline; 1.0560x over previous
import jax
import jax.numpy as jnp
from jax import lax
from jax.experimental import pallas as pl
from jax.experimental.pallas import tpu as pltpu

N_Z = 4
N_H = N_Z - 1
S = 2


def kernel(x):
    m_per, n = x.shape
    half = m_per // 2
    sub = half // S

    def body(x_ref, out_ref, copy_sem,
             sr_send, sr_recv,
             sl_send, sl_recv,
             sxr_send, sxr_recv,
             sxl_send, sxl_recv):
        my_x = lax.axis_index("x")
        my_y = lax.axis_index("y")
        my_z = lax.axis_index("z")
        partner = 1 - my_x
        hoff = my_x * half
        poff = partner * half

        def rows(c, off, s):
            return pl.ds(c * m_per + off + s * sub, sub)

        barrier = pltpu.get_barrier_semaphore()
        for k in range(1, N_Z):
            pl.semaphore_signal(
                barrier, inc=1, device_id=(my_x, my_y, (my_z + k) % N_Z),
                device_id_type=pl.DeviceIdType.MESH)
        pl.semaphore_signal(
            barrier, inc=1, device_id=(partner, my_y, my_z),
            device_id_type=pl.DeviceIdType.MESH)
        pl.semaphore_wait(barrier, N_Z)

        local = pltpu.make_async_copy(
            x_ref, out_ref.at[pl.ds(my_z * m_per, m_per), :], copy_sem)
        local.start()

        for h in range(N_H):
            cR = my_z - h
            cL = my_z + h
            aR = my_z - 1 - h
            aL = my_z + 1 + h
            cRc, cLc = cR % N_Z, cL % N_Z
            aRc, aLc = aR % N_Z, aL % N_Z
            send_r = (my_z < N_Z - 1) & (cR >= 0)
            send_l = (my_z > 0) & (cL <= N_Z - 1)
            recv_r = aR >= 0
            recv_l = aL <= N_Z - 1

            for s in range(S):
                @pl.when(send_r)
                def _(h=h, s=s, cRc=cRc):
                    if h == 0:
                        src = x_ref.at[pl.ds(hoff + s * sub, sub), :]
                    else:
                        src = out_ref.at[rows(cRc, hoff, s), :]
                    pltpu.make_async_remote_copy(
                        src_ref=src,
                        dst_ref=out_ref.at[rows(cRc, hoff, s), :],
                        send_sem=sr_send.at[h, s],
                        recv_sem=sr_recv.at[h, s],
                        device_id=(my_x, my_y, my_z + 1),
                        device_id_type=pl.DeviceIdType.MESH,
                    ).start()

                @pl.when(send_l)
                def _(h=h, s=s, cLc=cLc):
                    if h == 0:
                        src = x_ref.at[pl.ds(hoff + s * sub, sub), :]
                    else:
                        src = out_ref.at[rows(cLc, hoff, s), :]
                    pltpu.make_async_remote_copy(
                        src_ref=src,
                        dst_ref=out_ref.at[rows(cLc, hoff, s), :],
                        send_sem=sl_send.at[h, s],
                        recv_sem=sl_recv.at[h, s],
                        device_id=(my_x, my_y, my_z - 1),
                        device_id_type=pl.DeviceIdType.MESH,
                    ).start()

                @pl.when(recv_r)
                def _(h=h, s=s, aRc=aRc):
                    pltpu.make_async_remote_copy(
                        src_ref=out_ref.at[rows(aRc, hoff, s), :],
                        dst_ref=out_ref.at[rows(aRc, hoff, s), :],
                        send_sem=sr_send.at[h, s],
                        recv_sem=sr_recv.at[h, s],
                        device_id=(my_x, my_y, my_z - 1),
                        device_id_type=pl.DeviceIdType.MESH,
                    ).wait_recv()
                    pltpu.make_async_remote_copy(
                        src_ref=out_ref.at[rows(aRc, hoff, s), :],
                        dst_ref=out_ref.at[rows(aRc, hoff, s), :],
                        send_sem=sxr_send.at[h, s],
                        recv_sem=sxr_recv.at[h, s],
                        device_id=(partner, my_y, my_z),
                        device_id_type=pl.DeviceIdType.MESH,
                    ).start()

                @pl.when(recv_l)
                def _(h=h, s=s, aLc=aLc):
                    pltpu.make_async_remote_copy(
                        src_ref=out_ref.at[rows(aLc, hoff, s), :],
                        dst_ref=out_ref.at[rows(aLc, hoff, s), :],
                        send_sem=sl_send.at[h, s],
                        recv_sem=sl_recv.at[h, s],
                        device_id=(my_x, my_y, my_z + 1),
                        device_id_type=pl.DeviceIdType.MESH,
                    ).wait_recv()
                    pltpu.make_async_remote_copy(
                        src_ref=out_ref.at[rows(aLc, hoff, s), :],
                        dst_ref=out_ref.at[rows(aLc, hoff, s), :],
                        send_sem=sxl_send.at[h, s],
                        recv_sem=sxl_recv.at[h, s],
                        device_id=(partner, my_y, my_z),
                        device_id_type=pl.DeviceIdType.MESH,
                    ).start()

        for h in range(N_H):
            aR = my_z - 1 - h
            aL = my_z + 1 + h
            aRc, aLc = aR % N_Z, aL % N_Z
            had_r = aR >= 0
            had_l = aL <= N_Z - 1
            sent_r = (my_z < N_Z - 1) & (my_z - h >= 0)
            sent_l = (my_z > 0) & (my_z + h <= N_Z - 1)
            for s in range(S):
                @pl.when(had_r)
                def _(h=h, s=s, aRc=aRc):
                    pltpu.make_async_remote_copy(
                        src_ref=out_ref.at[rows(aRc, poff, s), :],
                        dst_ref=out_ref.at[rows(aRc, poff, s), :],
                        send_sem=sxr_send.at[h, s],
                        recv_sem=sxr_recv.at[h, s],
                        device_id=(partner, my_y, my_z),
                        device_id_type=pl.DeviceIdType.MESH,
                    ).wait_recv()
                    pltpu.make_async_remote_copy(
                        src_ref=out_ref.at[rows(aRc, hoff, s), :],
                        dst_ref=out_ref.at[rows(aRc, hoff, s), :],
                        send_sem=sxr_send.at[h, s],
                        recv_sem=sxr_recv.at[h, s],
                        device_id=(partner, my_y, my_z),
                        device_id_type=pl.DeviceIdType.MESH,
                    ).wait_send()

                @pl.when(had_l)
                def _(h=h, s=s, aLc=aLc):
                    pltpu.make_async_remote_copy(
                        src_ref=out_ref.at[rows(aLc, poff, s), :],
                        dst_ref=out_ref.at[rows(aLc, poff, s), :],
                        send_sem=sxl_send.at[h, s],
                        recv_sem=sxl_recv.at[h, s],
                        device_id=(partner, my_y, my_z),
                        device_id_type=pl.DeviceIdType.MESH,
                    ).wait_recv()
                    pltpu.make_async_remote_copy(
                        src_ref=out_ref.at[rows(aLc, hoff, s), :],
                        dst_ref=out_ref.at[rows(aLc, hoff, s), :],
                        send_sem=sxl_send.at[h, s],
                        recv_sem=sxl_recv.at[h, s],
                        device_id=(partner, my_y, my_z),
                        device_id_type=pl.DeviceIdType.MESH,
                    ).wait_send()

                @pl.when(sent_r)
                def _(h=h, s=s):
                    pltpu.make_async_remote_copy(
                        src_ref=out_ref.at[rows(0, hoff, s), :],
                        dst_ref=out_ref.at[rows(0, hoff, s), :],
                        send_sem=sr_send.at[h, s],
                        recv_sem=sr_recv.at[h, s],
                        device_id=(my_x, my_y, my_z + 1),
                        device_id_type=pl.DeviceIdType.MESH,
                    ).wait_send()

                @pl.when(sent_l)
                def _(h=h, s=s):
                    pltpu.make_async_remote_copy(
                        src_ref=out_ref.at[rows(0, hoff, s), :],
                        dst_ref=out_ref.at[rows(0, hoff, s), :],
                        send_sem=sl_send.at[h, s],
                        recv_sem=sl_recv.at[h, s],
                        device_id=(my_x, my_y, my_z - 1),
                        device_id_type=pl.DeviceIdType.MESH,
                    ).wait_send()

        local.wait()

    out_shape = jax.ShapeDtypeStruct((N_Z * m_per, n), x.dtype)
    sem_grid = pltpu.SemaphoreType.DMA((N_H, S))
    return pl.pallas_call(
        body,
        out_shape=out_shape,
        in_specs=[pl.BlockSpec(memory_space=pl.ANY)],
        out_specs=pl.BlockSpec(memory_space=pl.ANY),
        scratch_shapes=[
            pltpu.SemaphoreType.DMA,
            sem_grid, sem_grid,
            sem_grid, sem_grid,
            sem_grid, sem_grid,
            sem_grid, sem_grid,
        ],
        compiler_params=pltpu.CompilerParams(collective_id=0),
    )(x)


# device time: 2210879 ns/iter; 1.0587x vs baseline; 1.0025x over previous
import jax
import jax.numpy as jnp
from jax import lax
from jax.experimental import pallas as pl
from jax.experimental.pallas import tpu as pltpu

N_Z = 4
N_H = N_Z - 1
S = 4


def kernel(x):
    m_per, n = x.shape
    half = m_per // 2
    sub = half // S

    def body(x_ref, out_ref, copy_sem,
             sr_send, sr_recv,
             sl_send, sl_recv,
             sxr_send, sxr_recv,
             sxl_send, sxl_recv):
        my_x = lax.axis_index("x")
        my_y = lax.axis_index("y")
        my_z = lax.axis_index("z")
        partner = 1 - my_x
        hoff = my_x * half
        poff = partner * half

        def rows(c, off, s):
            return pl.ds(c * m_per + off + s * sub, sub)

        barrier = pltpu.get_barrier_semaphore()
        for k in range(1, N_Z):
            pl.semaphore_signal(
                barrier, inc=1, device_id=(my_x, my_y, (my_z + k) % N_Z),
                device_id_type=pl.DeviceIdType.MESH)
        pl.semaphore_signal(
            barrier, inc=1, device_id=(partner, my_y, my_z),
            device_id_type=pl.DeviceIdType.MESH)
        pl.semaphore_wait(barrier, N_Z)

        local = pltpu.make_async_copy(
            x_ref, out_ref.at[pl.ds(my_z * m_per, m_per), :], copy_sem)
        local.start()

        for h in range(N_H):
            cR = my_z - h
            cL = my_z + h
            aR = my_z - 1 - h
            aL = my_z + 1 + h
            cRc, cLc = cR % N_Z, cL % N_Z
            aRc, aLc = aR % N_Z, aL % N_Z
            send_r = (my_z < N_Z - 1) & (cR >= 0)
            send_l = (my_z > 0) & (cL <= N_Z - 1)
            recv_r = aR >= 0
            recv_l = aL <= N_Z - 1

            for s in range(S):
                @pl.when(send_r)
                def _(h=h, s=s, cRc=cRc):
                    if h == 0:
                        src = x_ref.at[pl.ds(hoff + s * sub, sub), :]
                    else:
                        src = out_ref.at[rows(cRc, hoff, s), :]
                    pltpu.make_async_remote_copy(
                        src_ref=src,
                        dst_ref=out_ref.at[rows(cRc, hoff, s), :],
                        send_sem=sr_send.at[h, s],
                        recv_sem=sr_recv.at[h, s],
                        device_id=(my_x, my_y, my_z + 1),
                        device_id_type=pl.DeviceIdType.MESH,
                    ).start()

                @pl.when(send_l)
                def _(h=h, s=s, cLc=cLc):
                    if h == 0:
                        src = x_ref.at[pl.ds(hoff + s * sub, sub), :]
                    else:
                        src = out_ref.at[rows(cLc, hoff, s), :]
                    pltpu.make_async_remote_copy(
                        src_ref=src,
                        dst_ref=out_ref.at[rows(cLc, hoff, s), :],
                        send_sem=sl_send.at[h, s],
                        recv_sem=sl_recv.at[h, s],
                        device_id=(my_x, my_y, my_z - 1),
                        device_id_type=pl.DeviceIdType.MESH,
                    ).start()

                @pl.when(recv_r)
                def _(h=h, s=s, aRc=aRc):
                    pltpu.make_async_remote_copy(
                        src_ref=out_ref.at[rows(aRc, hoff, s), :],
                        dst_ref=out_ref.at[rows(aRc, hoff, s), :],
                        send_sem=sr_send.at[h, s],
                        recv_sem=sr_recv.at[h, s],
                        device_id=(my_x, my_y, my_z - 1),
                        device_id_type=pl.DeviceIdType.MESH,
                    ).wait_recv()
                    pltpu.make_async_remote_copy(
                        src_ref=out_ref.at[rows(aRc, hoff, s), :],
                        dst_ref=out_ref.at[rows(aRc, hoff, s), :],
                        send_sem=sxr_send.at[h, s],
                        recv_sem=sxr_recv.at[h, s],
                        device_id=(partner, my_y, my_z),
                        device_id_type=pl.DeviceIdType.MESH,
                    ).start()

                @pl.when(recv_l)
                def _(h=h, s=s, aLc=aLc):
                    pltpu.make_async_remote_copy(
                        src_ref=out_ref.at[rows(aLc, hoff, s), :],
                        dst_ref=out_ref.at[rows(aLc, hoff, s), :],
                        send_sem=sl_send.at[h, s],
                        recv_sem=sl_recv.at[h, s],
                        device_id=(my_x, my_y, my_z + 1),
                        device_id_type=pl.DeviceIdType.MESH,
                    ).wait_recv()
                    pltpu.make_async_remote_copy(
                        src_ref=out_ref.at[rows(aLc, hoff, s), :],
                        dst_ref=out_ref.at[rows(aLc, hoff, s), :],
                        send_sem=sxl_send.at[h, s],
                        recv_sem=sxl_recv.at[h, s],
                        device_id=(partner, my_y, my_z),
                        device_id_type=pl.DeviceIdType.MESH,
                    ).start()

        for h in range(N_H):
            aR = my_z - 1 - h
            aL = my_z + 1 + h
            aRc, aLc = aR % N_Z, aL % N_Z
            had_r = aR >= 0
            had_l = aL <= N_Z - 1
            sent_r = (my_z < N_Z - 1) & (my_z - h >= 0)
            sent_l = (my_z > 0) & (my_z + h <= N_Z - 1)
            for s in range(S):
                @pl.when(had_r)
                def _(h=h, s=s, aRc=aRc):
                    pltpu.make_async_remote_copy(
                        src_ref=out_ref.at[rows(aRc, poff, s), :],
                        dst_ref=out_ref.at[rows(aRc, poff, s), :],
                        send_sem=sxr_send.at[h, s],
                        recv_sem=sxr_recv.at[h, s],
                        device_id=(partner, my_y, my_z),
                        device_id_type=pl.DeviceIdType.MESH,
                    ).wait_recv()
                    pltpu.make_async_remote_copy(
                        src_ref=out_ref.at[rows(aRc, hoff, s), :],
                        dst_ref=out_ref.at[rows(aRc, hoff, s), :],
                        send_sem=sxr_send.at[h, s],
                        recv_sem=sxr_recv.at[h, s],
                        device_id=(partner, my_y, my_z),
                        device_id_type=pl.DeviceIdType.MESH,
                    ).wait_send()

                @pl.when(had_l)
                def _(h=h, s=s, aLc=aLc):
                    pltpu.make_async_remote_copy(
                        src_ref=out_ref.at[rows(aLc, poff, s), :],
                        dst_ref=out_ref.at[rows(aLc, poff, s), :],
                        send_sem=sxl_send.at[h, s],
                        recv_sem=sxl_recv.at[h, s],
                        device_id=(partner, my_y, my_z),
                        device_id_type=pl.DeviceIdType.MESH,
                    ).wait_recv()
                    pltpu.make_async_remote_copy(
                        src_ref=out_ref.at[rows(aLc, hoff, s), :],
                        dst_ref=out_ref.at[rows(aLc, hoff, s), :],
                        send_sem=sxl_send.at[h, s],
                        recv_sem=sxl_recv.at[h, s],
                        device_id=(partner, my_y, my_z),
                        device_id_type=pl.DeviceIdType.MESH,
                    ).wait_send()

                @pl.when(sent_r)
                def _(h=h, s=s):
                    pltpu.make_async_remote_copy(
                        src_ref=out_ref.at[rows(0, hoff, s), :],
                        dst_ref=out_ref.at[rows(0, hoff, s), :],
                        send_sem=sr_send.at[h, s],
                        recv_sem=sr_recv.at[h, s],
                        device_id=(my_x, my_y, my_z + 1),
                        device_id_type=pl.DeviceIdType.MESH,
                    ).wait_send()

                @pl.when(sent_l)
                def _(h=h, s=s):
                    pltpu.make_async_remote_copy(
                        src_ref=out_ref.at[rows(0, hoff, s), :],
                        dst_ref=out_ref.at[rows(0, hoff, s), :],
                        send_sem=sl_send.at[h, s],
                        recv_sem=sl_recv.at[h, s],
                        device_id=(my_x, my_y, my_z - 1),
                        device_id_type=pl.DeviceIdType.MESH,
                    ).wait_send()

        local.wait()

    out_shape = jax.ShapeDtypeStruct((N_Z * m_per, n), x.dtype)
    sem_grid = pltpu.SemaphoreType.DMA((N_H, S))
    return pl.pallas_call(
        body,
        out_shape=out_shape,
        in_specs=[pl.BlockSpec(memory_space=pl.ANY)],
        out_specs=pl.BlockSpec(memory_space=pl.ANY),
        scratch_shapes=[
            pltpu.SemaphoreType.DMA,
            sem_grid, sem_grid,
            sem_grid, sem_grid,
            sem_grid, sem_grid,
            sem_grid, sem_grid,
        ],
        compiler_params=pltpu.CompilerParams(collective_id=0),
    )(x)
